# baseline (speedup 1.0000x reference)
import jax
import jax.numpy as jnp
from jax import lax
from jax.experimental import pallas as pl
from jax.experimental.pallas import tpu as pltpu


_BR = 64
_NBUF = 8


def _body(steps_ref, tab_ref, out_ref, buf_ref, sem_ref):
    tab = tab_ref[0, :]
    t = tab.shape[0]
    b, c, h, w = out_ref.shape
    n = b // _BR
    copies = [None] * n
    for i in range(n):
        k = i % _NBUF
        if i >= _NBUF:
            copies[i - _NBUF].wait()
        steps_c = steps_ref[pl.ds(i * _BR, _BR), :]
        lane = lax.broadcasted_iota(jnp.int32, (_BR, t), 1)
        eq = lane == steps_c
        vals = jnp.sum(jnp.where(eq, tab[None, :], 0.0), axis=1, keepdims=True)
        buf_ref[k] = jnp.broadcast_to(
            vals[:, :, None, None], (_BR, c, h, w)
        )
        copies[i] = pltpu.make_async_copy(
            buf_ref.at[k], out_ref.at[pl.ds(i * _BR, _BR)], sem_ref.at[k]
        )
        copies[i].start()
    for i in range(n - _NBUF, n):
        copies[i].wait()


def kernel(img, num_steps, alpha_bars):
    b, c, h, w = img.shape
    t_pad = 1024
    tab = jnp.zeros((1, t_pad), jnp.float32).at[0, : alpha_bars.shape[0]].set(
        alpha_bars
    )
    steps_col = num_steps.reshape(b, 1)

    return pl.pallas_call(
        _body,
        in_specs=[
            pl.BlockSpec(memory_space=pltpu.VMEM),
            pl.BlockSpec(memory_space=pltpu.VMEM),
        ],
        out_specs=pl.BlockSpec(memory_space=pl.ANY),
        out_shape=jax.ShapeDtypeStruct((b, c, h, w), jnp.float32),
        scratch_shapes=[
            pltpu.VMEM((_NBUF, _BR, c, h, w), jnp.float32),
            pltpu.SemaphoreType.DMA((_NBUF,)),
        ],
    )(steps_col, tab)


# speedup vs baseline: 5.2027x; 5.2027x over previous
import jax
import jax.numpy as jnp
from jax import lax
from jax.experimental import pallas as pl
from jax.experimental.pallas import tpu as pltpu


_BH = 16


def _body(steps_ref, tab_ref, out_ref, vals_ref):
    @pl.when((pl.program_id(0) == 0) & (pl.program_id(1) == 0))
    def _gather():
        steps = steps_ref[...]
        tab = tab_ref[...]
        t = tab.shape[0]
        b = steps.shape[1]
        sub = lax.broadcasted_iota(jnp.int32, (t, b), 0)
        eq = sub == steps
        vals_ref[...] = jnp.sum(
            jnp.where(eq, tab, 0.0), axis=0, keepdims=True
        )

    vals = vals_ref[...]
    out_ref[...] = jnp.broadcast_to(
        vals[:, None, None, :], out_ref.shape
    )


def kernel(img, num_steps, alpha_bars):
    b, c, h, w = img.shape
    t_pad = 1024
    tab_col = jnp.zeros((t_pad, 1), jnp.float32).at[: alpha_bars.shape[0], 0].set(
        alpha_bars
    )
    steps_row = num_steps.reshape(1, b)

    out_t = pl.pallas_call(
        _body,
        grid=(c, h // _BH),
        in_specs=[
            pl.BlockSpec((1, b), lambda i, j: (0, 0)),
            pl.BlockSpec((t_pad, 1), lambda i, j: (0, 0)),
        ],
        out_specs=pl.BlockSpec((1, _BH, w, b), lambda i, j: (i, j, 0, 0)),
        out_shape=jax.ShapeDtypeStruct((c, h, w, b), jnp.float32),
        scratch_shapes=[pltpu.VMEM((1, b), jnp.float32)],
    )(steps_row, tab_col)
    return jnp.transpose(out_t, (3, 0, 1, 2))


# speedup vs baseline: 5.3020x; 1.0191x over previous
import jax
import jax.numpy as jnp
from jax import lax
from jax.experimental import pallas as pl
from jax.experimental.pallas import tpu as pltpu


_BH = 16


def _body(steps_ref, tab_ref, out_ref, buf_ref, sem_ref):
    steps = steps_ref[...]
    tab = tab_ref[...]
    t = tab.shape[0]
    b = steps.shape[1]
    sub = lax.broadcasted_iota(jnp.int32, (t, b), 0)
    eq = sub == steps
    vals = jnp.sum(jnp.where(eq, tab, 0.0), axis=0, keepdims=True)
    buf_ref[...] = jnp.broadcast_to(vals[None, :, :], buf_ref.shape)

    c, h, w, _ = out_ref.shape
    nj = h // _BH
    copies = []
    for ci in range(c):
        for j in range(nj):
            cp = pltpu.make_async_copy(
                buf_ref,
                out_ref.at[ci, pl.ds(j * _BH, _BH)],
                sem_ref.at[ci * nj + j],
            )
            cp.start()
            copies.append(cp)
    for cp in copies:
        cp.wait()


def kernel(img, num_steps, alpha_bars):
    b, c, h, w = img.shape
    t_pad = 1024
    tab_col = jnp.zeros((t_pad, 1), jnp.float32).at[: alpha_bars.shape[0], 0].set(
        alpha_bars
    )
    steps_row = num_steps.reshape(1, b)
    ndma = c * (h // _BH)

    out_t = pl.pallas_call(
        _body,
        in_specs=[
            pl.BlockSpec(memory_space=pltpu.VMEM),
            pl.BlockSpec(memory_space=pltpu.VMEM),
        ],
        out_specs=pl.BlockSpec(memory_space=pl.ANY),
        out_shape=jax.ShapeDtypeStruct((c, h, w, b), jnp.float32),
        scratch_shapes=[
            pltpu.VMEM((_BH, w, b), jnp.float32),
            pltpu.SemaphoreType.DMA((ndma,)),
        ],
    )(steps_row, tab_col)
    return jnp.transpose(out_t, (3, 0, 1, 2))


# speedup vs baseline: 5.3157x; 1.0026x over previous
import jax
import jax.numpy as jnp
from jax import lax
from jax.experimental import pallas as pl
from jax.experimental.pallas import tpu as pltpu


_BH = 8


def _body(steps_ref, tab_ref, out_ref, buf_ref, sem_ref):
    steps = steps_ref[...]
    tab = tab_ref[...]
    t = tab.shape[0]
    b = steps.shape[1]
    sub = lax.broadcasted_iota(jnp.int32, (t, b), 0)
    eq = sub == steps
    vals = jnp.sum(jnp.where(eq, tab, 0.0), axis=0, keepdims=True)
    buf_ref[...] = jnp.broadcast_to(vals[None, :, :], buf_ref.shape)

    c, h, w, _ = out_ref.shape
    nj = h // _BH
    copies = []
    for ci in range(c):
        for j in range(nj):
            cp = pltpu.make_async_copy(
                buf_ref,
                out_ref.at[ci, pl.ds(j * _BH, _BH)],
                sem_ref.at[ci * nj + j],
            )
            cp.start()
            copies.append(cp)
    for cp in copies:
        cp.wait()


def kernel(img, num_steps, alpha_bars):
    b, c, h, w = img.shape
    t_pad = 1024
    tab_col = jnp.zeros((t_pad, 1), jnp.float32).at[: alpha_bars.shape[0], 0].set(
        alpha_bars
    )
    steps_row = num_steps.reshape(1, b)
    ndma = c * (h // _BH)

    out_t = pl.pallas_call(
        _body,
        in_specs=[
            pl.BlockSpec(memory_space=pltpu.VMEM),
            pl.BlockSpec(memory_space=pltpu.VMEM),
        ],
        out_specs=pl.BlockSpec(memory_space=pl.ANY),
        out_shape=jax.ShapeDtypeStruct((c, h, w, b), jnp.float32),
        scratch_shapes=[
            pltpu.VMEM((_BH, w, b), jnp.float32),
            pltpu.SemaphoreType.DMA((ndma,)),
        ],
    )(steps_row, tab_col)
    return jnp.transpose(out_t, (3, 0, 1, 2))
